# Initial kernel scaffold; baseline (speedup 1.0000x reference)
import jax
import jax.numpy as jnp
from jax.experimental import pallas as pl


def kernel(x):
    raise NotImplementedError("write your pallas kernel here")


# speedup vs baseline: 7.8986x; 7.8986x over previous
import jax
import jax.numpy as jnp
from jax.experimental import pallas as pl

_W = 4
_K = 12


def _tc_body(x_ref, o_ref):
    xb = x_ref[...]
    t = xb.shape[-1]
    tw = t - _W + 1
    w = (xb[..., 0:tw] + xb[..., 1:tw + 1] + xb[..., 2:tw + 2]
         + xb[..., 3:tw + 3])
    cur = w * (1.0 / _W)
    acc = jnp.zeros(cur.shape[:-1], jnp.float32)
    rem = jnp.full(cur.shape[:-1], _K, jnp.int32)
    neg = jnp.float32(-jnp.inf)
    for _ in range(_K):
        v = jnp.max(cur, axis=-1)
        eq = cur >= v[..., None]
        cnt = jnp.sum(eq.astype(jnp.int32), axis=-1)
        take = jnp.minimum(cnt, rem)
        acc = acc + jnp.where(take > 0, v * take.astype(jnp.float32), 0.0)
        rem = rem - take
        cur = jnp.where(eq, neg, cur)
    z_cf = acc * (1.0 / _K)
    o_ref[...] = jnp.mean(z_cf, axis=-1, keepdims=True)[..., None]


def kernel(x):
    b, c, f, t = x.shape
    if _W <= 1 or t < _W:
        return x.mean(axis=(-1, -2))
    xr = x.reshape(b * c, f, t)
    out = pl.pallas_call(
        _tc_body,
        grid=(b * c,),
        in_specs=[pl.BlockSpec((1, f, t), lambda i: (i, 0, 0))],
        out_specs=pl.BlockSpec((1, 1, 1), lambda i: (i, 0, 0)),
        out_shape=jax.ShapeDtypeStruct((b * c, 1, 1), jnp.float32),
    )(xr)
    return out.reshape(b, c)


# speedup vs baseline: 8.2272x; 1.0416x over previous
import functools

import jax
import jax.numpy as jnp
from jax import lax
from jax.experimental import pallas as pl
from jax.experimental.pallas import tpu as pltpu
from jax.experimental.pallas import tpu_sc as plsc

_W = 4
_K = 12
_NC = 2
_NS = 16
_L = 16


def _sort(v, descending=False):
    r = plsc.sort_key_val(v, v, descending=descending)
    if isinstance(r, (tuple, list)):
        r = r[0]
    return r


def _sc_pool_topk(groups, f, t):
    rows = groups * f
    nw = _NC * _NS
    rows_pw = rows // nw
    groups_pw = groups // nw
    ch = 64
    nvr = t // _L
    tw = t - _W + 1
    last_valid = tw - (nvr - 1) * _L
    scale = 1.0 / (_W * _K * f)
    neg = jnp.float32(-jnp.inf)

    mesh = plsc.VectorSubcoreMesh(
        core_axis_name="c", subcore_axis_name="s",
        num_cores=_NC, num_subcores=_NS,
    )

    @functools.partial(
        pl.kernel,
        mesh=mesh,
        out_type=jax.ShapeDtypeStruct((groups,), jnp.float32),
        compiler_params=pltpu.CompilerParams(needs_layout_passes=False),
        scratch_types=[
            pltpu.VMEM((ch * t + _L,), jnp.float32),
            pltpu.VMEM((ch * t + _L,), jnp.float32),
            pltpu.VMEM((groups_pw,), jnp.float32),
            pltpu.SemaphoreType.DMA,
            pltpu.SemaphoreType.DMA,
        ],
    )
    def run(x_hbm, out_hbm, xbuf0, xbuf1, resbuf, sem0, sem1):
        wid = lax.axis_index("s") * _NC + lax.axis_index("c")
        row0 = wid * rows_pw
        lane = lax.iota(jnp.int32, _L)
        topmask = lane >= (_L - _K)
        lastmask = lane < last_valid

        def start_chunk(chunk_idx, buf, sem):
            src = x_hbm.at[pl.ds((row0 + chunk_idx * ch) * t, ch * t)]
            pltpu.make_async_copy(src, buf.at[pl.ds(0, ch * t)], sem).start()

        start_chunk(0, xbuf0, sem0)
        start_chunk(1, xbuf1, sem1)

        def row_topk(buf, r):
            base = r * t

            def cand(i, R):
                off = base + i * _L
                a0 = buf[pl.ds(off, _L)]
                a1 = buf[pl.ds(off + 1, _L)]
                a2 = buf[pl.ds(off + 2, _L)]
                a3 = buf[pl.ds(off + 3, _L)]
                ws = (a0 + a1) + (a2 + a3)
                c_dsc = _sort(ws, descending=True)
                return _sort(jnp.maximum(R, c_dsc))

            R = lax.fori_loop(0, nvr - 1, cand, jnp.full((_L,), neg))
            off = base + (nvr - 1) * _L
            a0 = buf[pl.ds(off, _L)]
            a1 = buf[pl.ds(off + 1, _L)]
            a2 = buf[pl.ds(off + 2, _L)]
            a3 = buf[pl.ds(off + 3, _L)]
            ws = jnp.where(lastmask, (a0 + a1) + (a2 + a3), neg)
            c_dsc = _sort(ws, descending=True)
            R = _sort(jnp.maximum(R, c_dsc))
            return jnp.where(topmask, R, 0.0)

        def chunk_sum(buf, gacc):
            def row_body(r, acc):
                return acc + row_topk(buf, r)
            return lax.fori_loop(0, ch, row_body, gacc)

        def group_body(g, _):
            pltpu.make_async_copy(
                x_hbm.at[pl.ds(0, ch * t)], xbuf0.at[pl.ds(0, ch * t)], sem0
            ).wait()
            gacc = chunk_sum(xbuf0, jnp.zeros((_L,), jnp.float32))

            @pl.when(g + 1 < groups_pw)
            def _():
                start_chunk(2 * (g + 1), xbuf0, sem0)

            pltpu.make_async_copy(
                x_hbm.at[pl.ds(0, ch * t)], xbuf1.at[pl.ds(0, ch * t)], sem1
            ).wait()
            gacc = chunk_sum(xbuf1, gacc)

            @pl.when(g + 1 < groups_pw)
            def _():
                start_chunk(2 * (g + 1) + 1, xbuf1, sem1)

            tot = plsc.cumsum(gacc) * scale
            plsc.store_scatter(
                resbuf, [jnp.full((_L,), g, jnp.int32)], tot,
                mask=lane == (_L - 1),
            )
            return 0

        lax.fori_loop(0, groups_pw, group_body, 0)
        pltpu.sync_copy(resbuf, out_hbm.at[pl.ds(wid * groups_pw, groups_pw)])

    return run


def kernel(x):
    b, c, f, t = x.shape
    if _W <= 1 or t < _W:
        return x.mean(axis=(-1, -2))
    xr = x.reshape(b * c * f * t)
    out = _sc_pool_topk(b * c, f, t)(xr)
    return out.reshape(b, c)


# speedup vs baseline: 22.1207x; 2.6887x over previous
import functools

import jax
import jax.numpy as jnp
from jax import lax
from jax.experimental import pallas as pl
from jax.experimental.pallas import tpu as pltpu
from jax.experimental.pallas import tpu_sc as plsc

_W = 4
_K = 12
_NC = 2
_NS = 16
_L = 16


def _sort(v, descending=False):
    r = plsc.sort_key_val(v, v, descending=descending)
    if isinstance(r, (tuple, list)):
        r = r[0]
    return r


def _sc_pool_topk(groups, f, t):
    rows = groups * f
    nw = _NC * _NS
    rows_pw = rows // nw
    groups_pw = groups // nw
    ch = 64
    nvr = t // _L
    tw = t - _W + 1
    last_valid = tw - (nvr - 1) * _L
    scale = 1.0 / (_W * _K * f)
    neg = jnp.float32(-jnp.inf)

    mesh = plsc.VectorSubcoreMesh(
        core_axis_name="c", subcore_axis_name="s",
        num_cores=_NC, num_subcores=_NS,
    )

    @functools.partial(
        pl.kernel,
        mesh=mesh,
        out_type=jax.ShapeDtypeStruct((groups,), jnp.float32),
        compiler_params=pltpu.CompilerParams(needs_layout_passes=False),
        scratch_types=[
            pltpu.VMEM((ch * t + _L,), jnp.float32),
            pltpu.VMEM((ch * t + _L,), jnp.float32),
            pltpu.VMEM((groups_pw,), jnp.float32),
            pltpu.SemaphoreType.DMA,
            pltpu.SemaphoreType.DMA,
        ],
    )
    def run(x_hbm, out_hbm, xbuf0, xbuf1, resbuf, sem0, sem1):
        wid = lax.axis_index("s") * _NC + lax.axis_index("c")
        row0 = wid * rows_pw
        lane = lax.iota(jnp.int32, _L)
        topmask = lane >= (_L - _K)
        lastmask = lane < last_valid

        def start_chunk(chunk_idx, buf, sem):
            src = x_hbm.at[pl.ds((row0 + chunk_idx * ch) * t, ch * t)]
            pltpu.make_async_copy(src, buf.at[pl.ds(0, ch * t)], sem).start()

        start_chunk(0, xbuf0, sem0)
        start_chunk(1, xbuf1, sem1)

        ilv = 8

        def _wsum(buf, off):
            a0 = buf[pl.ds(off, _L)]
            a1 = buf[pl.ds(off + 1, _L)]
            a2 = buf[pl.ds(off + 2, _L)]
            a3 = buf[pl.ds(off + 3, _L)]
            return (a0 + a1) + (a2 + a3)

        def chunk_sum(buf, gacc):
            def blk(q, acc):
                r0 = q * ilv

                def cand(i, Rs):
                    ioff = i * _L
                    out = []
                    for j in range(ilv):
                        ws = _wsum(buf, (r0 + j) * t + ioff)
                        c_dsc = _sort(ws, descending=True)
                        out.append(_sort(jnp.maximum(Rs[j], c_dsc)))
                    return tuple(out)

                Rs = lax.fori_loop(
                    0, nvr - 1, cand,
                    tuple(jnp.full((_L,), neg) for _ in range(ilv)),
                )
                ioff = (nvr - 1) * _L
                for j in range(ilv):
                    ws = jnp.where(
                        lastmask, _wsum(buf, (r0 + j) * t + ioff), neg
                    )
                    c_dsc = _sort(ws, descending=True)
                    R = _sort(jnp.maximum(Rs[j], c_dsc))
                    acc = acc + jnp.where(topmask, R, 0.0)
                return acc

            return lax.fori_loop(0, ch // ilv, blk, gacc)

        def group_body(g, _):
            pltpu.make_async_copy(
                x_hbm.at[pl.ds(0, ch * t)], xbuf0.at[pl.ds(0, ch * t)], sem0
            ).wait()
            gacc = chunk_sum(xbuf0, jnp.zeros((_L,), jnp.float32))

            @pl.when(g + 1 < groups_pw)
            def _():
                start_chunk(2 * (g + 1), xbuf0, sem0)

            pltpu.make_async_copy(
                x_hbm.at[pl.ds(0, ch * t)], xbuf1.at[pl.ds(0, ch * t)], sem1
            ).wait()
            gacc = chunk_sum(xbuf1, gacc)

            @pl.when(g + 1 < groups_pw)
            def _():
                start_chunk(2 * (g + 1) + 1, xbuf1, sem1)

            tot = plsc.cumsum(gacc) * scale
            plsc.store_scatter(
                resbuf, [jnp.full((_L,), g, jnp.int32)], tot,
                mask=lane == (_L - 1),
            )
            return 0

        lax.fori_loop(0, groups_pw, group_body, 0)
        pltpu.sync_copy(resbuf, out_hbm.at[pl.ds(wid * groups_pw, groups_pw)])

    return run


def kernel(x):
    b, c, f, t = x.shape
    if _W <= 1 or t < _W:
        return x.mean(axis=(-1, -2))
    xr = x.reshape(b * c * f * t)
    out = _sc_pool_topk(b * c, f, t)(xr)
    return out.reshape(b, c)
